# baseline (device time: 13449 ns/iter reference)
import jax
import jax.numpy as jnp
from jax import lax
from jax.experimental import pallas as pl
from jax.experimental.pallas import tpu as pltpu

N_DEV = 4
BLK = 256


def kernel(x, w_mat):
    k_full, k_shard = x.shape
    n = w_mat.shape[1]

    def body(x_ref, w_hbm, out_ref, xs_ref, xg_ref, wv_ref,
             send_sems, recv_sems, w_sem):
        my = lax.axis_index("i")

        w_copy = pltpu.make_async_copy(w_hbm, wv_ref, w_sem)
        w_copy.start()

        xs_ref[:, :] = x_ref[:, :].astype(jnp.bfloat16)

        barrier_sem = pltpu.get_barrier_semaphore()
        for d in range(1, N_DEV):
            peer = lax.rem(my + d, N_DEV)
            pl.semaphore_signal(
                barrier_sem, inc=1,
                device_id=(peer,), device_id_type=pl.DeviceIdType.MESH,
            )
        pl.semaphore_wait(barrier_sem, N_DEV - 1)

        rdmas = []
        for d in range(1, N_DEV):
            dst = lax.rem(my + d, N_DEV)
            rdma = pltpu.make_async_remote_copy(
                src_ref=xs_ref.at[pl.ds(dst * BLK, BLK), :],
                dst_ref=xg_ref.at[my],
                send_sem=send_sems.at[d - 1],
                recv_sem=recv_sems.at[my],
                device_id=(dst,),
                device_id_type=pl.DeviceIdType.MESH,
            )
            rdma.start()
            rdmas.append(rdma)

        for j in range(N_DEV):
            @pl.when(my == j)
            def _(j=j):
                xg_ref[j] = xs_ref[j * BLK:(j + 1) * BLK, :]

        w_copy.wait()

        acc = jnp.zeros((BLK, n), dtype=jnp.float32)
        for j in range(N_DEV):
            @pl.when(my != j)
            def _(j=j):
                recv = pltpu.make_async_remote_copy(
                    src_ref=xs_ref.at[pl.ds(0, BLK), :],
                    dst_ref=xg_ref.at[j],
                    send_sem=send_sems.at[0],
                    recv_sem=recv_sems.at[j],
                    device_id=(my,),
                    device_id_type=pl.DeviceIdType.MESH,
                )
                recv.wait_recv()
            acc += jnp.dot(
                xg_ref[j],
                wv_ref[j * BLK:(j + 1) * BLK, :].astype(jnp.bfloat16),
                preferred_element_type=jnp.float32,
            )

        out_ref[:, :] = acc * jax.nn.sigmoid(acc)

        for rdma in rdmas:
            rdma.wait_send()

    return pl.pallas_call(
        body,
        out_shape=jax.ShapeDtypeStruct((BLK, n), jnp.float32),
        in_specs=[
            pl.BlockSpec(memory_space=pltpu.VMEM),
            pl.BlockSpec(memory_space=pl.ANY),
        ],
        out_specs=pl.BlockSpec(memory_space=pltpu.VMEM),
        scratch_shapes=[
            pltpu.VMEM((k_full, k_shard), jnp.bfloat16),
            pltpu.VMEM((N_DEV, BLK, BLK), jnp.bfloat16),
            pltpu.VMEM((k_full, n), jnp.float32),
            pltpu.SemaphoreType.DMA((N_DEV - 1,)),
            pltpu.SemaphoreType.DMA((N_DEV,)),
            pltpu.SemaphoreType.DMA,
        ],
        compiler_params=pltpu.CompilerParams(collective_id=0),
    )(x, w_mat)
